# baseline (speedup 1.0000x reference)
import functools

import jax
import jax.numpy as jnp
from jax import lax
from jax.experimental import pallas as pl
from jax.experimental.pallas import tpu as pltpu
from jax.experimental.pallas import tpu_sc as plsc

N = 10000
E = 320000
T = 2
D = 128
G = 64

NC = 2
NS = 16
NW = NC * NS
NPAD = 10240
CH = 128
NCH = 80
EPAD = NW * NCH * CH
RPT = NPAD // NS
GP = 80
GCH = NPAD // CH
IB = 8
SCH = 64
TCH = EPAD // SCH
BLKT = TCH // (NS * IB)
BLK0 = 40
BLK1 = BLKT - BLK0
NBUF = 4

BN = 512
GRID = NPAD // BN


def _seg_body(xc_hbm, src_hbm, dst_hbm, zh_hbm,
              out_h, src_v, dst_v, rows_v, hsh, sem0, sem1, sem2, sem3):
    c = lax.axis_index("c")
    s = lax.axis_index("s")
    pltpu.sync_copy(zh_hbm, hsh.at[pl.ds(s * RPT, RPT)])
    plsc.subcore_barrier()

    sems = (sem0, sem1, sem2, sem3)

    def run(cbase, nblocks):
        @pl.loop(0, nblocks)
        def _(jb):
            start = cbase + jb * IB
            pltpu.sync_copy(src_hbm.at[pl.ds(start, IB)], src_v)
            pltpu.sync_copy(dst_hbm.at[pl.ds(start, IB)], dst_v)
            cps = [None] * NBUF
            for k in range(NBUF - 1):
                cps[k] = pltpu.async_copy(
                    xc_hbm.at[src_v.at[k]], rows_v.at[k], sems[k])
            for k in range(IB):
                b = k % NBUF
                if k + NBUF - 1 < IB:
                    b2 = (k + NBUF - 1) % NBUF
                    cps[b2] = pltpu.async_copy(
                        xc_hbm.at[src_v.at[k + NBUF - 1]], rows_v.at[b2],
                        sems[b2])
                cps[b].wait()
                pltpu.sync_copy(rows_v.at[b], hsh.at[dst_v.at[k]], add=True)

    @pl.when(c == 0)
    def _():
        run(s * (BLK0 * IB), BLK0)

    @pl.when(c == 1)
    def _():
        run(NS * (BLK0 * IB) + s * (BLK1 * IB), BLK1)

    plsc.subcore_barrier()
    pltpu.sync_copy(hsh.at[pl.ds(s * RPT, RPT)],
                    out_h.at[c, pl.ds(s * RPT, RPT)])


def _cnt_body(dst_hbm, zh_hbm, ones_hbm,
              out_c, dst_v, ones_v, csh):
    c = lax.axis_index("c")
    s = lax.axis_index("s")
    wid = s * NC + c
    pltpu.sync_copy(zh_hbm, csh.at[pl.ds(s * RPT, RPT)])
    pltpu.sync_copy(ones_hbm, ones_v)
    plsc.subcore_barrier()

    @pl.loop(0, NCH // IB)
    def _(jb):
        pltpu.sync_copy(dst_hbm.at[wid, pl.ds(jb * IB, IB)], dst_v)
        for k in range(IB):
            pltpu.sync_copy(ones_v, csh.at[dst_v.at[k]], add=True)

    plsc.subcore_barrier()
    pltpu.sync_copy(csh.at[pl.ds(s * RPT, RPT)],
                    out_c.at[c, pl.ds(s * RPT, RPT)])


def _pool_body(e2_hbm, off_hbm, zg_hbm,
               out_g, off_v, rows_v, gsh):
    c = lax.axis_index("c")
    s = lax.axis_index("s")
    wid = s * NC + c

    @pl.when(s == 0)
    def _():
        pltpu.sync_copy(zg_hbm, gsh)

    plsc.subcore_barrier()

    for k in range(3):
        cid = wid + NW * k

        @pl.when(cid < GCH)
        def _():
            pltpu.sync_copy(e2_hbm.at[pl.ds(cid * CH, CH)], rows_v)
            pltpu.sync_copy(off_hbm.at[cid], off_v.at[0])
            pltpu.sync_copy(rows_v, gsh.at[off_v.at[0]], add=True)

    plsc.subcore_barrier()

    @pl.when(s == 0)
    def _():
        pltpu.sync_copy(gsh, out_g.at[c])


@functools.lru_cache(maxsize=None)
def _build_segsum():
    mesh = plsc.VectorSubcoreMesh(core_axis_name="c", subcore_axis_name="s")
    return pl.kernel(
        _seg_body,
        out_type=jax.ShapeDtypeStruct((NC, NPAD, D), jnp.float32),
        mesh=mesh,
        scratch_types=[
            pltpu.VMEM((IB, SCH), jnp.int32),
            pltpu.VMEM((IB, SCH), jnp.int32),
            pltpu.VMEM((NBUF, SCH, D), jnp.float32),
            pltpu.VMEM_SHARED((NPAD, D), jnp.float32),
            pltpu.SemaphoreType.DMA,
            pltpu.SemaphoreType.DMA,
            pltpu.SemaphoreType.DMA,
            pltpu.SemaphoreType.DMA,
        ],
    )


@functools.lru_cache(maxsize=None)
def _build_cnt():
    mesh = plsc.VectorSubcoreMesh(core_axis_name="c", subcore_axis_name="s")
    return pl.kernel(
        _cnt_body,
        out_type=jax.ShapeDtypeStruct((NC, NPAD, D), jnp.float32),
        mesh=mesh,
        scratch_types=[
            pltpu.VMEM((IB, CH), jnp.int32),
            pltpu.VMEM((CH, D), jnp.float32),
            pltpu.VMEM_SHARED((NPAD, D), jnp.float32),
        ],
    )


@functools.lru_cache(maxsize=None)
def _build_pool():
    mesh = plsc.VectorSubcoreMesh(core_axis_name="c", subcore_axis_name="s")
    return pl.kernel(
        _pool_body,
        out_type=jax.ShapeDtypeStruct((NC, GP, D), jnp.float32),
        mesh=mesh,
        scratch_types=[
            pltpu.VMEM((1, CH), jnp.int32),
            pltpu.VMEM((CH, D), jnp.float32),
            pltpu.VMEM_SHARED((GP, D), jnp.float32),
        ],
    )


def _segsum_sc(*args):
    return _build_segsum()(*args)


def _cnt_sc(*args):
    return _build_cnt()(*args)


def _pool_sc(*args):
    return _build_pool()(*args)


def _lin_body(x_ref, w_ref, b_ref, o_ref):
    i = pl.program_id(0)
    rows = lax.broadcasted_iota(jnp.int32, (BN, D), 0) + i * BN
    out = jnp.dot(x_ref[...], w_ref[...],
                  preferred_element_type=jnp.float32) + b_ref[...]
    o_ref[...] = jnp.where(rows < N, out, 0.0)


def _linear(x, w_t, b):
    return pl.pallas_call(
        _lin_body,
        grid=(GRID,),
        in_specs=[
            pl.BlockSpec((BN, D), lambda i: (i, 0)),
            pl.BlockSpec((D, D), lambda i: (0, 0)),
            pl.BlockSpec((1, D), lambda i: (0, 0)),
        ],
        out_specs=pl.BlockSpec((BN, D), lambda i: (i, 0)),
        out_shape=jax.ShapeDtypeStruct((NPAD, D), jnp.float32),
    )(x, w_t, b)


def _gru_body(mode, xc_ref, h0_ref, h1_ref, c0_ref, c1_ref,
              wih_ref, whh_ref, bih_ref, bhh_ref, *rest):
    o_ref = rest[-1]
    xv = xc_ref[...]
    hs = h0_ref[...] + h1_ref[...]
    cnt = c0_ref[...][:, 0:1] + c1_ref[...][:, 0:1]
    h = hs / jnp.maximum(cnt, 1.0)
    gi = jnp.dot(xv, wih_ref[...], preferred_element_type=jnp.float32)
    gi = gi + bih_ref[...]
    gh = jnp.dot(h, whh_ref[...], preferred_element_type=jnp.float32)
    gh = gh + bhh_ref[...]
    r = jax.nn.sigmoid(gi[:, :D] + gh[:, :D])
    z = jax.nn.sigmoid(gi[:, D:2 * D] + gh[:, D:2 * D])
    n = jnp.tanh(gi[:, 2 * D:] + r * gh[:, 2 * D:])
    out = (1.0 - z) * n + z * h
    out = jnp.where(h == 0.0, xv, out)
    if mode in ("relu", "relu_lin"):
        out = jnp.maximum(out, 0.0)
    if mode == "relu_lin":
        w2_ref, b2_ref = rest[0], rest[1]
        out = jnp.dot(out, w2_ref[...],
                      preferred_element_type=jnp.float32) + b2_ref[...]
    o_ref[...] = out


def _gru(mode, xc, h_p, c_p, wih_t, whh_t, bih, bhh, w2_t=None, b2=None):
    full = lambda shape: pl.BlockSpec(shape, lambda i: (0, 0))
    row = pl.BlockSpec((BN, D), lambda i: (i, 0))
    rowc = pl.BlockSpec((BN, 8), lambda i: (i, 0))
    in_specs = [row, row, row, rowc, rowc,
                full((D, 3 * D)), full((D, 3 * D)),
                full((1, 3 * D)), full((1, 3 * D))]
    args = [xc, h_p[0], h_p[1], c_p[0], c_p[1], wih_t, whh_t, bih, bhh]
    if mode == "relu_lin":
        in_specs += [full((D, D)), full((1, D))]
        args += [w2_t, b2]
    return pl.pallas_call(
        functools.partial(_gru_body, mode),
        grid=(GRID,),
        in_specs=in_specs,
        out_specs=row,
        out_shape=jax.ShapeDtypeStruct((NPAD, D), jnp.float32),
    )(*args)


def _fin_body(g_ref, off_ref, o_ref):
    gsum = g_ref[0, :G, :] + g_ref[1, :G, :]
    offs = off_ref[...]
    gids = lax.broadcasted_iota(jnp.int32, (G, GCH, CH), 0)
    cnt = jnp.sum(jnp.where(offs[None, :, :] == gids, 1.0, 0.0),
                  axis=(1, 2), keepdims=False)
    o_ref[...] = gsum / jnp.maximum(cnt[:, None], 1.0)


def _finalize(g_p, off_pad):
    return pl.pallas_call(
        _fin_body,
        out_shape=jax.ShapeDtypeStruct((G, D), jnp.float32),
    )(g_p, off_pad)


def kernel(x, offset, edge, W1, b1, W2, b2, W_ih, W_hh, b_ih, b_hh):
    w1t = W1.T
    w2t = W2.T
    wiht = W_ih.T
    whht = W_hh.T
    b1r = b1.reshape(1, D)
    b2r = b2.reshape(1, D)
    bihr = b_ih.reshape(1, 3 * D)
    bhhr = b_hh.reshape(1, 3 * D)

    pad_idx = jnp.full((EPAD - E,), N, dtype=jnp.int32)
    edges = []
    dsts = []
    for t in range(T):
        src = jnp.concatenate([edge[t, 0], pad_idx]).reshape(TCH, SCH)
        dst = jnp.concatenate([edge[t, 1], pad_idx]).reshape(TCH, SCH)
        edges.append((src, dst))
        dsts.append(jnp.concatenate([edge[t, 1], pad_idx]).reshape(NW, NCH, CH))

    off_pad = jnp.concatenate(
        [offset.astype(jnp.int32), jnp.full((NPAD - N,), G, dtype=jnp.int32)]
    ).reshape(GCH, CH)

    zh = jnp.zeros((RPT, D), jnp.float32)
    zg = jnp.zeros((GP, D), jnp.float32)
    ones = jnp.ones((CH, D), jnp.float32)

    cnts = [_cnt_sc(dsts[t], zh, ones)[:, :, :8] for t in range(T)]

    xc = _linear(x, w1t, b1r)
    for conv in range(2):
        for t in range(T):
            src, dst = edges[t]
            h_p = _segsum_sc(xc, src, dst, zh)
            last = t == T - 1
            if conv == 0 and last:
                mode = "relu_lin"
            elif conv == 1 and last:
                mode = "relu"
            else:
                mode = "plain"
            if mode == "relu_lin":
                xc = _gru(mode, xc, h_p, cnts[t], wiht, whht, bihr, bhhr,
                          w2t, b2r)
            else:
                xc = _gru(mode, xc, h_p, cnts[t], wiht, whht, bihr, bhhr)

    g_p = _pool_sc(xc, off_pad, zg)
    return _finalize(g_p, off_pad)


# speedup vs baseline: 1.3201x; 1.3201x over previous
import functools

import jax
import jax.numpy as jnp
from jax import lax
from jax.experimental import pallas as pl
from jax.experimental.pallas import tpu as pltpu
from jax.experimental.pallas import tpu_sc as plsc

N = 10000
E = 320000
T = 2
D = 128
G = 64

NC = 2
NS = 16
NW = NC * NS
NPAD = 10240
CH = 128
NCH = 80
EPAD = NW * NCH * CH
RPT = NPAD // NS
GP = 80
GCH = NPAD // CH
IB = 8
SCH = 64
TCH = EPAD // SCH
BLKT = TCH // (NS * IB)
BLK0 = 32
BLK1 = BLKT - BLK0
NBUF = 4

BN = 512
GRID = NPAD // BN


def _seg_body(xc_hbm, src_hbm, dst_hbm, zh_hbm,
              out_h, src_v, dst_v, rows_v, hsh, sem0, sem1, sem2, sem3):
    c = lax.axis_index("c")
    s = lax.axis_index("s")
    pltpu.sync_copy(zh_hbm, hsh.at[pl.ds(s * RPT, RPT)])
    plsc.subcore_barrier()

    sems = (sem0, sem1, sem2, sem3)

    def run(cbase, nblocks):
        @pl.loop(0, nblocks)
        def _(jb):
            start = cbase + jb * IB
            pltpu.sync_copy(src_hbm.at[pl.ds(start, IB)], src_v)
            pltpu.sync_copy(dst_hbm.at[pl.ds(start, IB)], dst_v)
            cps = [None] * NBUF
            for k in range(NBUF - 1):
                cps[k] = pltpu.async_copy(
                    xc_hbm.at[src_v.at[k]], rows_v.at[k], sems[k])
            for k in range(IB):
                b = k % NBUF
                if k + NBUF - 1 < IB:
                    b2 = (k + NBUF - 1) % NBUF
                    cps[b2] = pltpu.async_copy(
                        xc_hbm.at[src_v.at[k + NBUF - 1]], rows_v.at[b2],
                        sems[b2])
                cps[b].wait()
                pltpu.sync_copy(rows_v.at[b], hsh.at[dst_v.at[k]], add=True)

    @pl.when(c == 0)
    def _():
        run(s * (BLK0 * IB), BLK0)

    @pl.when(c == 1)
    def _():
        run(NS * (BLK0 * IB) + s * (BLK1 * IB), BLK1)

    plsc.subcore_barrier()
    pltpu.sync_copy(hsh.at[pl.ds(s * RPT, RPT)],
                    out_h.at[c, pl.ds(s * RPT, RPT)])


def _cnt_body(dst_hbm, zh_hbm, ones_hbm,
              out_c, dst_v, ones_v, csh):
    c = lax.axis_index("c")
    s = lax.axis_index("s")
    wid = s * NC + c
    pltpu.sync_copy(zh_hbm, csh.at[pl.ds(s * RPT, RPT)])
    pltpu.sync_copy(ones_hbm, ones_v)
    plsc.subcore_barrier()

    @pl.loop(0, NCH // IB)
    def _(jb):
        pltpu.sync_copy(dst_hbm.at[wid, pl.ds(jb * IB, IB)], dst_v)
        for k in range(IB):
            pltpu.sync_copy(ones_v, csh.at[dst_v.at[k]], add=True)

    plsc.subcore_barrier()
    pltpu.sync_copy(csh.at[pl.ds(s * RPT, RPT)],
                    out_c.at[c, pl.ds(s * RPT, RPT)])


def _pool_body(e2_hbm, off_hbm, zg_hbm,
               out_g, off_v, rows_v, gsh):
    c = lax.axis_index("c")
    s = lax.axis_index("s")
    wid = s * NC + c

    @pl.when(s == 0)
    def _():
        pltpu.sync_copy(zg_hbm, gsh)

    plsc.subcore_barrier()

    for k in range(3):
        cid = wid + NW * k

        @pl.when(cid < GCH)
        def _():
            pltpu.sync_copy(e2_hbm.at[pl.ds(cid * CH, CH)], rows_v)
            pltpu.sync_copy(off_hbm.at[cid], off_v.at[0])
            pltpu.sync_copy(rows_v, gsh.at[off_v.at[0]], add=True)

    plsc.subcore_barrier()

    @pl.when(s == 0)
    def _():
        pltpu.sync_copy(gsh, out_g.at[c])


@functools.lru_cache(maxsize=None)
def _build_segsum():
    mesh = plsc.VectorSubcoreMesh(core_axis_name="c", subcore_axis_name="s")
    return pl.kernel(
        _seg_body,
        out_type=jax.ShapeDtypeStruct((NC, NPAD, D), jnp.float32),
        mesh=mesh,
        scratch_types=[
            pltpu.VMEM((IB, SCH), jnp.int32),
            pltpu.VMEM((IB, SCH), jnp.int32),
            pltpu.VMEM((NBUF, SCH, D), jnp.float32),
            pltpu.VMEM_SHARED((NPAD, D), jnp.float32),
            pltpu.SemaphoreType.DMA,
            pltpu.SemaphoreType.DMA,
            pltpu.SemaphoreType.DMA,
            pltpu.SemaphoreType.DMA,
        ],
    )


@functools.lru_cache(maxsize=None)
def _build_cnt():
    mesh = plsc.VectorSubcoreMesh(core_axis_name="c", subcore_axis_name="s")
    return pl.kernel(
        _cnt_body,
        out_type=jax.ShapeDtypeStruct((NC, NPAD, D), jnp.float32),
        mesh=mesh,
        scratch_types=[
            pltpu.VMEM((IB, CH), jnp.int32),
            pltpu.VMEM((CH, D), jnp.float32),
            pltpu.VMEM_SHARED((NPAD, D), jnp.float32),
        ],
    )


@functools.lru_cache(maxsize=None)
def _build_pool():
    mesh = plsc.VectorSubcoreMesh(core_axis_name="c", subcore_axis_name="s")
    return pl.kernel(
        _pool_body,
        out_type=jax.ShapeDtypeStruct((NC, GP, D), jnp.float32),
        mesh=mesh,
        scratch_types=[
            pltpu.VMEM((1, CH), jnp.int32),
            pltpu.VMEM((CH, D), jnp.float32),
            pltpu.VMEM_SHARED((GP, D), jnp.float32),
        ],
    )


def _segsum_sc(*args):
    return _build_segsum()(*args)


def _cnt_sc(*args):
    return _build_cnt()(*args)


def _pool_sc(*args):
    return _build_pool()(*args)


def _lin_body(x_ref, w_ref, b_ref, o_ref):
    i = pl.program_id(0)
    rows = lax.broadcasted_iota(jnp.int32, (BN, D), 0) + i * BN
    out = jnp.dot(x_ref[...], w_ref[...],
                  preferred_element_type=jnp.float32) + b_ref[...]
    o_ref[...] = jnp.where(rows < N, out, 0.0)


def _linear(x, w_t, b):
    return pl.pallas_call(
        _lin_body,
        grid=(GRID,),
        in_specs=[
            pl.BlockSpec((BN, D), lambda i: (i, 0)),
            pl.BlockSpec((D, D), lambda i: (0, 0)),
            pl.BlockSpec((1, D), lambda i: (0, 0)),
        ],
        out_specs=pl.BlockSpec((BN, D), lambda i: (i, 0)),
        out_shape=jax.ShapeDtypeStruct((NPAD, D), jnp.float32),
    )(x, w_t, b)


def _gru_body(mode, xc_ref, h0_ref, h1_ref, c0_ref, c1_ref,
              wih_ref, whh_ref, bih_ref, bhh_ref, *rest):
    o_ref = rest[-1]
    xv = xc_ref[...]
    hs = h0_ref[...] + h1_ref[...]
    cnt = c0_ref[...][:, 0:1] + c1_ref[...][:, 0:1]
    h = hs / jnp.maximum(cnt, 1.0)
    gi = jnp.dot(xv, wih_ref[...], preferred_element_type=jnp.float32)
    gi = gi + bih_ref[...]
    gh = jnp.dot(h, whh_ref[...], preferred_element_type=jnp.float32)
    gh = gh + bhh_ref[...]
    r = jax.nn.sigmoid(gi[:, :D] + gh[:, :D])
    z = jax.nn.sigmoid(gi[:, D:2 * D] + gh[:, D:2 * D])
    n = jnp.tanh(gi[:, 2 * D:] + r * gh[:, 2 * D:])
    out = (1.0 - z) * n + z * h
    out = jnp.where(h == 0.0, xv, out)
    if mode in ("relu", "relu_lin"):
        out = jnp.maximum(out, 0.0)
    if mode == "relu_lin":
        w2_ref, b2_ref = rest[0], rest[1]
        out = jnp.dot(out, w2_ref[...],
                      preferred_element_type=jnp.float32) + b2_ref[...]
    o_ref[...] = out


def _gru(mode, xc, h_p, c_p, wih_t, whh_t, bih, bhh, w2_t=None, b2=None):
    full = lambda shape: pl.BlockSpec(shape, lambda i: (0, 0))
    row = pl.BlockSpec((BN, D), lambda i: (i, 0))
    rowc = pl.BlockSpec((BN, 8), lambda i: (i, 0))
    in_specs = [row, row, row, rowc, rowc,
                full((D, 3 * D)), full((D, 3 * D)),
                full((1, 3 * D)), full((1, 3 * D))]
    args = [xc, h_p[0], h_p[1], c_p[0], c_p[1], wih_t, whh_t, bih, bhh]
    if mode == "relu_lin":
        in_specs += [full((D, D)), full((1, D))]
        args += [w2_t, b2]
    return pl.pallas_call(
        functools.partial(_gru_body, mode),
        grid=(GRID,),
        in_specs=in_specs,
        out_specs=row,
        out_shape=jax.ShapeDtypeStruct((NPAD, D), jnp.float32),
    )(*args)


def _fin_body(g_ref, off_ref, o_ref):
    gsum = g_ref[0, :G, :] + g_ref[1, :G, :]
    offs = off_ref[...]
    gids = lax.broadcasted_iota(jnp.int32, (G, GCH, CH), 0)
    cnt = jnp.sum(jnp.where(offs[None, :, :] == gids, 1.0, 0.0),
                  axis=(1, 2), keepdims=False)
    o_ref[...] = gsum / jnp.maximum(cnt[:, None], 1.0)


def _finalize(g_p, off_pad):
    return pl.pallas_call(
        _fin_body,
        out_shape=jax.ShapeDtypeStruct((G, D), jnp.float32),
    )(g_p, off_pad)


def kernel(x, offset, edge, W1, b1, W2, b2, W_ih, W_hh, b_ih, b_hh):
    w1t = W1.T
    w2t = W2.T
    wiht = W_ih.T
    whht = W_hh.T
    b1r = b1.reshape(1, D)
    b2r = b2.reshape(1, D)
    bihr = b_ih.reshape(1, 3 * D)
    bhhr = b_hh.reshape(1, 3 * D)

    pad_idx = jnp.full((EPAD - E,), N, dtype=jnp.int32)
    edges = []
    dsts = []
    for t in range(T):
        src = jnp.concatenate([edge[t, 0], pad_idx]).reshape(TCH, SCH)
        dst = jnp.concatenate([edge[t, 1], pad_idx]).reshape(TCH, SCH)
        edges.append((src, dst))
        dsts.append(jnp.concatenate([edge[t, 1], pad_idx]).reshape(NW, NCH, CH))

    off_pad = jnp.concatenate(
        [offset.astype(jnp.int32), jnp.full((NPAD - N,), G, dtype=jnp.int32)]
    ).reshape(GCH, CH)

    zh = jnp.zeros((RPT, D), jnp.float32)
    zg = jnp.zeros((GP, D), jnp.float32)
    ones = jnp.ones((CH, D), jnp.float32)

    cnts = [_cnt_sc(dsts[t], zh, ones)[:, :, :8] for t in range(T)]

    xc = _linear(x, w1t, b1r)
    for conv in range(2):
        for t in range(T):
            src, dst = edges[t]
            h_p = _segsum_sc(xc, src, dst, zh)
            last = t == T - 1
            if conv == 0 and last:
                mode = "relu_lin"
            elif conv == 1 and last:
                mode = "relu"
            else:
                mode = "plain"
            if mode == "relu_lin":
                xc = _gru(mode, xc, h_p, cnts[t], wiht, whht, bihr, bhhr,
                          w2t, b2r)
            else:
                xc = _gru(mode, xc, h_p, cnts[t], wiht, whht, bihr, bhhr)

    g_p = _pool_sc(xc, off_pad, zg)
    return _finalize(g_p, off_pad)
